# baseline (speedup 1.0000x reference)
import jax
import jax.numpy as jnp
from jax import lax
from jax.experimental import pallas as pl
from jax.experimental.pallas import tpu as pltpu
from jax.experimental.pallas import tpu_sc as plsc

N = 10000
C = 128
E = 320000

NC = 2
NS = 16
CH = C // NC
ROWS_PER_TEC = N // NS
K = 128
NW = 160
CW = 16
NCHUNK = NW // CW
E_PAD = NS * NW * K
ACC_ROWS = N + 8
TABLE_MOD = 5

MMBLK = 1000


def _mm_body(x_ref, w_ref, o_ref):
    o_ref[...] = jnp.dot(x_ref[...], w_ref[0],
                         preferred_element_type=jnp.float32)[None]


def _matmul_split(x, w_stacked):
    return pl.pallas_call(
        _mm_body,
        grid=(NC, N // MMBLK),
        in_specs=[pl.BlockSpec((MMBLK, C), lambda c, i: (i, 0)),
                  pl.BlockSpec((1, C, CH), lambda c, i: (c, 0, 0))],
        out_specs=pl.BlockSpec((1, MMBLK, CH), lambda c, i: (c, i, 0)),
        out_shape=jax.ShapeDtypeStruct((NC, N, CH), jnp.float32),
    )(x, w_stacked)


def _stats_body(h_ref, g_ref, b_ref, o_ref, acc_ref):
    i = pl.program_id(0)

    @pl.when(i == 0)
    def _():
        acc_ref[...] = jnp.zeros_like(acc_ref)

    blk = h_ref[...]
    acc_ref[0:1, :] += jnp.sum(blk, axis=0, keepdims=True)
    acc_ref[1:2, :] += jnp.sum(blk * blk, axis=0, keepdims=True)

    mean = acc_ref[0:1, :] / N
    var = acc_ref[1:2, :] / N - mean * mean
    mul = g_ref[...] * lax.rsqrt(var + 1e-5)
    add = b_ref[...] - mean * mul
    o_ref[...] = jnp.concatenate([mul, add], axis=0)


def _bn_stats(h, gamma, beta):
    return pl.pallas_call(
        _stats_body,
        grid=(N // MMBLK,),
        in_specs=[pl.BlockSpec((MMBLK, C), lambda i: (i, 0)),
                  pl.BlockSpec((1, C), lambda i: (0, 0)),
                  pl.BlockSpec((1, C), lambda i: (0, 0))],
        out_specs=pl.BlockSpec((2, C), lambda i: (0, 0)),
        out_shape=jax.ShapeDtypeStruct((2, C), jnp.float32),
        scratch_shapes=[pltpu.VMEM((2, C), jnp.float32)],
    )(h, gamma.reshape(1, C), beta.reshape(1, C))


def _affine_relu_mm_body(h_ref, s_ref, w_ref, o_ref):
    t = jnp.maximum(h_ref[...] * s_ref[0:1, :] + s_ref[1:2, :], 0.0)
    o_ref[...] = jnp.dot(t, w_ref[0], preferred_element_type=jnp.float32)[None]


def _affine_relu_matmul_split(h, stats, w_stacked):
    return pl.pallas_call(
        _affine_relu_mm_body,
        grid=(NC, N // MMBLK),
        in_specs=[pl.BlockSpec((MMBLK, C), lambda c, i: (i, 0)),
                  pl.BlockSpec((2, C), lambda c, i: (0, 0)),
                  pl.BlockSpec((1, C, CH), lambda c, i: (c, 0, 0))],
        out_specs=pl.BlockSpec((1, MMBLK, CH), lambda c, i: (c, i, 0)),
        out_shape=jax.ShapeDtypeStruct((NC, N, CH), jnp.float32),
    )(h, stats, w_stacked)


def _final_body(h_ref, s_ref, x_ref, o_ref):
    o_ref[...] = jnp.maximum(
        h_ref[...] * s_ref[0:1, :] + s_ref[1:2, :] + x_ref[...], 0.0)


def _affine_residual_relu(h, stats, x):
    return pl.pallas_call(
        _final_body,
        grid=(N // MMBLK,),
        in_specs=[pl.BlockSpec((MMBLK, C), lambda i: (i, 0)),
                  pl.BlockSpec((2, C), lambda i: (0, 0)),
                  pl.BlockSpec((MMBLK, C), lambda i: (i, 0))],
        out_specs=pl.BlockSpec((MMBLK, C), lambda i: (i, 0)),
        out_shape=jax.ShapeDtypeStruct((N, C), jnp.float32),
    )(h, stats, x)


def _conv_sc_body(z_hbm, src_hbm, dst_hbm, out_hbm,
                  table_sp, accum, src_vm, dst_vm, bufs,
                  semgs, semss, semi):
    cid = lax.axis_index("c")
    sid = lax.axis_index("s")
    ch0 = cid * CH
    r0 = sid * ROWS_PER_TEC

    table = z_hbm.at[cid]

    pltpu.sync_copy(table.at[pl.ds(r0, ROWS_PER_TEC)],
                    table_sp.at[pl.ds(r0, ROWS_PER_TEC)])
    pltpu.sync_copy(table.at[pl.ds(r0, ROWS_PER_TEC)],
                    accum.at[pl.ds(r0, ROWS_PER_TEC)])

    pltpu.sync_copy(src_hbm.at[sid, pl.ds(0, CW)], src_vm.at[0])
    pltpu.sync_copy(dst_hbm.at[sid, pl.ds(0, CW)], dst_vm.at[0])

    plsc.subcore_barrier()

    def idx_fetch(nc, p):
        return (pltpu.make_async_copy(
                    src_hbm.at[sid, pl.ds(nc * CW, CW)], src_vm.at[p],
                    semi),
                pltpu.make_async_copy(
                    dst_hbm.at[sid, pl.ds(nc * CW, CW)], dst_vm.at[p],
                    semi))

    def gather(w, b):
        p = (w // CW) % 2
        row = w % CW
        idx = src_vm.at[p, row]
        g_hbm = pltpu.make_async_copy(table.at[idx], bufs[b], semgs[b])
        g_spm = pltpu.make_async_copy(table_sp.at[idx], bufs[b], semgs[b])
        use_sp = (row % TABLE_MOD) == (TABLE_MOD - 1)
        return g_hbm, g_spm, use_sp

    def gather_start(w, b):
        g_hbm, g_spm, use_sp = gather(w, b)

        @pl.when(use_sp)
        def _():
            g_spm.start()

        @pl.when(jnp.logical_not(use_sp))
        def _():
            g_hbm.start()

    def gather_wait(w, b):
        g_hbm, g_spm, use_sp = gather(w, b)

        @pl.when(use_sp)
        def _():
            g_spm.wait()

        @pl.when(jnp.logical_not(use_sp))
        def _():
            g_hbm.wait()

    def scatter(w, b):
        p = (w // CW) % 2
        row = w % CW
        return pltpu.make_async_copy(bufs[b], accum.at[dst_vm.at[p, row]],
                                     semss[b])

    gather_start(0, 0)
    gather_start(1, 1)

    def quad(base, q):
        for j in range(4):
            w = base + 4 * q + j
            b = j
            b2 = (j + 2) % 4
            gather_wait(w, b)
            scatter(w, b).start(add=True)

            @pl.when(w + 2 < NW)
            def _():
                @pl.when(w >= 2)
                def _():
                    scatter(w, b2).wait()

                gather_start(w + 2, b2)

    def chunk(nc, _):
        base = nc * CW

        @pl.when(nc + 1 < NCHUNK)
        def _():
            fs, fd = idx_fetch(nc + 1, (nc + 1) % 2)
            fs.start()
            fd.start()

        def quad_loop(q, _):
            quad(base, q)
            return 0

        lax.fori_loop(0, 3, quad_loop, 0)

        @pl.when(nc + 1 < NCHUNK)
        def _():
            fs, fd = idx_fetch(nc + 1, (nc + 1) % 2)
            fs.wait()
            fd.wait()

        quad(base, 3)
        return 0

    lax.fori_loop(0, NCHUNK, chunk, 0)
    for b in range(4):
        scatter(0, b).wait()

    plsc.subcore_barrier()

    pltpu.sync_copy(accum.at[pl.ds(r0, ROWS_PER_TEC)],
                    out_hbm.at[pl.ds(r0, ROWS_PER_TEC), pl.ds(ch0, CH)])


_CONV_SC_CACHE = []


def _conv_sc(z, src, dst):
    if not _CONV_SC_CACHE:
        _CONV_SC_CACHE.append(pl.kernel(
            _conv_sc_body,
            out_type=jax.ShapeDtypeStruct((N, C), jnp.float32),
            mesh=plsc.VectorSubcoreMesh(core_axis_name="c",
                                        subcore_axis_name="s"),
            scratch_types=[
                pltpu.VMEM_SHARED((N, CH), jnp.float32),
                pltpu.VMEM_SHARED((ACC_ROWS, CH), jnp.float32),
                pltpu.VMEM((2, CW, K), jnp.int32),
                pltpu.VMEM((2, CW, K), jnp.int32),
                [pltpu.VMEM((K, CH), jnp.float32) for _ in range(4)],
                [pltpu.SemaphoreType.DMA for _ in range(4)],
                [pltpu.SemaphoreType.DMA for _ in range(4)],
                pltpu.SemaphoreType.DMA,
            ],
            compiler_params=pltpu.CompilerParams(use_tc_tiling_on_sc=False),
        ))
    return _CONV_SC_CACHE[0](z, src, dst)


def kernel(x, edge_index, W1, gamma1, beta1, W2, gamma2, beta2):
    npad = E_PAD - E
    pad_i = jnp.arange(npad, dtype=jnp.int32)
    src = jnp.concatenate([edge_index[0], (pad_i * 131) % N]).reshape(
        NS, NW, K)
    dst = jnp.concatenate([edge_index[1], N + (pad_i % 8)]).reshape(NS, NW, K)

    w1s = jnp.stack([W1[:, :CH], W1[:, CH:]])
    w2s = jnp.stack([W2[:, :CH], W2[:, CH:]])

    z1 = _matmul_split(x, w1s)
    h = _conv_sc(z1, src, dst)
    stats1 = _bn_stats(h, gamma1, beta1)
    z2 = _affine_relu_matmul_split(h, stats1, w2s)
    h2 = _conv_sc(z2, src, dst)
    stats2 = _bn_stats(h2, gamma2, beta2)
    return _affine_residual_relu(h2, stats2, x)


# speedup vs baseline: 1.0438x; 1.0438x over previous
import jax
import jax.numpy as jnp
from jax import lax
from jax.experimental import pallas as pl
from jax.experimental.pallas import tpu as pltpu
from jax.experimental.pallas import tpu_sc as plsc

N = 10000
C = 128
E = 320000

NC = 2
NS = 16
CH = C // NC
ROWS_PER_TEC = N // NS
K = 128
NW = 160
E_PAD = NS * NW * K
ACC_ROWS = N + 8

MMBLK = 1000


def _mm_body(x_ref, w_ref, o_ref):
    o_ref[...] = jnp.dot(x_ref[...], w_ref[0],
                         preferred_element_type=jnp.float32)[None]


def _matmul_split(x, w_stacked):
    return pl.pallas_call(
        _mm_body,
        grid=(NC, N // MMBLK),
        in_specs=[pl.BlockSpec((MMBLK, C), lambda c, i: (i, 0)),
                  pl.BlockSpec((1, C, CH), lambda c, i: (c, 0, 0))],
        out_specs=pl.BlockSpec((1, MMBLK, CH), lambda c, i: (c, i, 0)),
        out_shape=jax.ShapeDtypeStruct((NC, N, CH), jnp.float32),
    )(x, w_stacked)


def _stats_body(h_ref, g_ref, b_ref, o_ref, acc_ref):
    i = pl.program_id(0)

    @pl.when(i == 0)
    def _():
        acc_ref[...] = jnp.zeros_like(acc_ref)

    blk = h_ref[...]
    acc_ref[0:1, :] += jnp.sum(blk, axis=0, keepdims=True)
    acc_ref[1:2, :] += jnp.sum(blk * blk, axis=0, keepdims=True)

    mean = acc_ref[0:1, :] / N
    var = acc_ref[1:2, :] / N - mean * mean
    mul = g_ref[...] * lax.rsqrt(var + 1e-5)
    add = b_ref[...] - mean * mul
    o_ref[...] = jnp.concatenate([mul, add], axis=0)


def _bn_stats(h, gamma, beta):
    return pl.pallas_call(
        _stats_body,
        grid=(N // MMBLK,),
        in_specs=[pl.BlockSpec((MMBLK, C), lambda i: (i, 0)),
                  pl.BlockSpec((1, C), lambda i: (0, 0)),
                  pl.BlockSpec((1, C), lambda i: (0, 0))],
        out_specs=pl.BlockSpec((2, C), lambda i: (0, 0)),
        out_shape=jax.ShapeDtypeStruct((2, C), jnp.float32),
        scratch_shapes=[pltpu.VMEM((2, C), jnp.float32)],
    )(h, gamma.reshape(1, C), beta.reshape(1, C))


def _affine_relu_mm_body(h_ref, s_ref, w_ref, o_ref):
    t = jnp.maximum(h_ref[...] * s_ref[0:1, :] + s_ref[1:2, :], 0.0)
    o_ref[...] = jnp.dot(t, w_ref[0], preferred_element_type=jnp.float32)[None]


def _affine_relu_matmul_split(h, stats, w_stacked):
    return pl.pallas_call(
        _affine_relu_mm_body,
        grid=(NC, N // MMBLK),
        in_specs=[pl.BlockSpec((MMBLK, C), lambda c, i: (i, 0)),
                  pl.BlockSpec((2, C), lambda c, i: (0, 0)),
                  pl.BlockSpec((1, C, CH), lambda c, i: (c, 0, 0))],
        out_specs=pl.BlockSpec((1, MMBLK, CH), lambda c, i: (c, i, 0)),
        out_shape=jax.ShapeDtypeStruct((NC, N, CH), jnp.float32),
    )(h, stats, w_stacked)


def _final_body(h_ref, s_ref, x_ref, o_ref):
    o_ref[...] = jnp.maximum(
        h_ref[...] * s_ref[0:1, :] + s_ref[1:2, :] + x_ref[...], 0.0)


def _affine_residual_relu(h, stats, x):
    return pl.pallas_call(
        _final_body,
        grid=(N // MMBLK,),
        in_specs=[pl.BlockSpec((MMBLK, C), lambda i: (i, 0)),
                  pl.BlockSpec((2, C), lambda i: (0, 0)),
                  pl.BlockSpec((MMBLK, C), lambda i: (i, 0))],
        out_specs=pl.BlockSpec((MMBLK, C), lambda i: (i, 0)),
        out_shape=jax.ShapeDtypeStruct((N, C), jnp.float32),
    )(h, stats, x)


def _conv_sc_body(z_hbm, src_hbm, dst_hbm, out_hbm,
                  accum, src_vm, dst_vm, bufs, semgs, semss):
    cid = lax.axis_index("c")
    sid = lax.axis_index("s")
    ch0 = cid * CH
    r0 = sid * ROWS_PER_TEC

    table = z_hbm.at[cid]

    pltpu.sync_copy(table.at[pl.ds(r0, ROWS_PER_TEC)],
                    accum.at[pl.ds(r0, ROWS_PER_TEC)])

    pltpu.sync_copy(src_hbm.at[sid], src_vm)
    pltpu.sync_copy(dst_hbm.at[sid], dst_vm)

    plsc.subcore_barrier()

    def gather(w, b):
        return pltpu.make_async_copy(table.at[src_vm.at[w]], bufs[b],
                                     semgs[b])

    def scatter(w, b):
        return pltpu.make_async_copy(bufs[b], accum.at[dst_vm.at[w]],
                                     semss[b])

    gather(0, 0).start()
    gather(1, 1).start()
    for w in (0, 1):
        gather(w, w).wait()
        scatter(w, w).start(add=True)
        gather(w + 2, w + 2).start()

    def quad(i, _):
        for j in range(4):
            w = 2 + 4 * i + j
            b = (2 + j) % 4
            gather(w, b).wait()
            scatter(w, b).start(add=True)
            scatter(w, j).wait()
            gather(w + 2, j).start()
        return 0

    lax.fori_loop(0, (NW - 4) // 4, quad, 0)
    for w in (NW - 2, NW - 1):
        gather(w, w % 4).wait()
        scatter(w, w % 4).start(add=True)
    for b in range(4):
        scatter(0, b).wait()

    plsc.subcore_barrier()

    pltpu.sync_copy(accum.at[pl.ds(r0, ROWS_PER_TEC)],
                    out_hbm.at[pl.ds(r0, ROWS_PER_TEC), pl.ds(ch0, CH)])


_CONV_SC_CACHE = []


def _conv_sc(z, src, dst):
    if not _CONV_SC_CACHE:
        _CONV_SC_CACHE.append(pl.kernel(
            _conv_sc_body,
            out_type=jax.ShapeDtypeStruct((N, C), jnp.float32),
            mesh=plsc.VectorSubcoreMesh(core_axis_name="c",
                                        subcore_axis_name="s"),
            scratch_types=[
                pltpu.VMEM_SHARED((ACC_ROWS, CH), jnp.float32),
                pltpu.VMEM((NW, K), jnp.int32),
                pltpu.VMEM((NW, K), jnp.int32),
                [pltpu.VMEM((K, CH), jnp.float32) for _ in range(4)],
                [pltpu.SemaphoreType.DMA for _ in range(4)],
                [pltpu.SemaphoreType.DMA for _ in range(4)],
            ],
            compiler_params=pltpu.CompilerParams(use_tc_tiling_on_sc=False),
        ))
    return _CONV_SC_CACHE[0](z, src, dst)


def kernel(x, edge_index, W1, gamma1, beta1, W2, gamma2, beta2):
    npad = E_PAD - E
    pad_i = jnp.arange(npad, dtype=jnp.int32)
    src = jnp.concatenate([edge_index[0], (pad_i * 131) % N]).reshape(
        NS, NW, K)
    dst = jnp.concatenate([edge_index[1], N + (pad_i % 8)]).reshape(NS, NW, K)

    w1s = jnp.stack([W1[:, :CH], W1[:, CH:]])
    w2s = jnp.stack([W2[:, :CH], W2[:, CH:]])

    z1 = _matmul_split(x, w1s)
    h = _conv_sc(z1, src, dst)
    stats1 = _bn_stats(h, gamma1, beta1)
    z2 = _affine_relu_matmul_split(h, stats1, w2s)
    h2 = _conv_sc(z2, src, dst)
    stats2 = _bn_stats(h2, gamma2, beta2)
    return _affine_residual_relu(h2, stats2, x)


# speedup vs baseline: 1.1031x; 1.0568x over previous
import jax
import jax.numpy as jnp
from jax import lax
from jax.experimental import pallas as pl
from jax.experimental.pallas import tpu as pltpu
from jax.experimental.pallas import tpu_sc as plsc

N = 10000
C = 128
E = 320000

NC = 2
NS = 16
CH = C // NC
ROWS_PER_TEC = N // NS
K = 128
NW = 162
E_PAD = NS * NW * K
ACC_ROWS = N + 8

MMBLK = 1000


def _mm_body(x_ref, w_ref, o_ref):
    o_ref[...] = jnp.dot(x_ref[...], w_ref[0],
                         preferred_element_type=jnp.float32)[None]


def _matmul_split(x, w_stacked):
    return pl.pallas_call(
        _mm_body,
        grid=(NC, N // MMBLK),
        in_specs=[pl.BlockSpec((MMBLK, C), lambda c, i: (i, 0)),
                  pl.BlockSpec((1, C, CH), lambda c, i: (c, 0, 0))],
        out_specs=pl.BlockSpec((1, MMBLK, CH), lambda c, i: (c, i, 0)),
        out_shape=jax.ShapeDtypeStruct((NC, N, CH), jnp.float32),
    )(x, w_stacked)


def _stats_body(h_ref, g_ref, b_ref, o_ref, acc_ref):
    i = pl.program_id(0)

    @pl.when(i == 0)
    def _():
        acc_ref[...] = jnp.zeros_like(acc_ref)

    blk = h_ref[...]
    acc_ref[0:1, :] += jnp.sum(blk, axis=0, keepdims=True)
    acc_ref[1:2, :] += jnp.sum(blk * blk, axis=0, keepdims=True)

    mean = acc_ref[0:1, :] / N
    var = acc_ref[1:2, :] / N - mean * mean
    mul = g_ref[...] * lax.rsqrt(var + 1e-5)
    add = b_ref[...] - mean * mul
    o_ref[...] = jnp.concatenate([mul, add], axis=0)


def _bn_stats(h, gamma, beta):
    return pl.pallas_call(
        _stats_body,
        grid=(N // MMBLK,),
        in_specs=[pl.BlockSpec((MMBLK, C), lambda i: (i, 0)),
                  pl.BlockSpec((1, C), lambda i: (0, 0)),
                  pl.BlockSpec((1, C), lambda i: (0, 0))],
        out_specs=pl.BlockSpec((2, C), lambda i: (0, 0)),
        out_shape=jax.ShapeDtypeStruct((2, C), jnp.float32),
        scratch_shapes=[pltpu.VMEM((2, C), jnp.float32)],
    )(h, gamma.reshape(1, C), beta.reshape(1, C))


def _affine_relu_mm_body(h_ref, s_ref, w_ref, o_ref):
    t = jnp.maximum(h_ref[...] * s_ref[0:1, :] + s_ref[1:2, :], 0.0)
    o_ref[...] = jnp.dot(t, w_ref[0], preferred_element_type=jnp.float32)[None]


def _affine_relu_matmul_split(h, stats, w_stacked):
    return pl.pallas_call(
        _affine_relu_mm_body,
        grid=(NC, N // MMBLK),
        in_specs=[pl.BlockSpec((MMBLK, C), lambda c, i: (i, 0)),
                  pl.BlockSpec((2, C), lambda c, i: (0, 0)),
                  pl.BlockSpec((1, C, CH), lambda c, i: (c, 0, 0))],
        out_specs=pl.BlockSpec((1, MMBLK, CH), lambda c, i: (c, i, 0)),
        out_shape=jax.ShapeDtypeStruct((NC, N, CH), jnp.float32),
    )(h, stats, w_stacked)


def _final_body(h_ref, s_ref, x_ref, o_ref):
    o_ref[...] = jnp.maximum(
        h_ref[...] * s_ref[0:1, :] + s_ref[1:2, :] + x_ref[...], 0.0)


def _affine_residual_relu(h, stats, x):
    return pl.pallas_call(
        _final_body,
        grid=(N // MMBLK,),
        in_specs=[pl.BlockSpec((MMBLK, C), lambda i: (i, 0)),
                  pl.BlockSpec((2, C), lambda i: (0, 0)),
                  pl.BlockSpec((MMBLK, C), lambda i: (i, 0))],
        out_specs=pl.BlockSpec((MMBLK, C), lambda i: (i, 0)),
        out_shape=jax.ShapeDtypeStruct((N, C), jnp.float32),
    )(h, stats, x)


def _conv_sc_body(z_hbm, src_hbm, dst_hbm, out_hbm,
                  accum, src_vm, dst_vm, bufs, semgs, semss):
    cid = lax.axis_index("c")
    sid = lax.axis_index("s")
    ch0 = cid * CH
    r0 = sid * ROWS_PER_TEC

    table = z_hbm.at[cid]

    pltpu.sync_copy(table.at[pl.ds(r0, ROWS_PER_TEC)],
                    accum.at[pl.ds(r0, ROWS_PER_TEC)])

    pltpu.sync_copy(src_hbm.at[sid], src_vm)
    pltpu.sync_copy(dst_hbm.at[sid], dst_vm)

    plsc.subcore_barrier()

    def gather(w, b):
        return pltpu.make_async_copy(table.at[src_vm.at[w]], bufs[b],
                                     semgs[b])

    def scatter(w, b):
        return pltpu.make_async_copy(bufs[b], accum.at[dst_vm.at[w]],
                                     semss[b])

    for w in (0, 1, 2):
        gather(w, w).start()
    for w in (0, 1, 2):
        gather(w, w).wait()
        scatter(w, w).start(add=True)
        gather(w + 3, w + 3).start()

    def sextet(i, _):
        for j in range(6):
            w = 3 + 6 * i + j
            b = (3 + j) % 6
            gather(w, b).wait()
            scatter(w, b).start(add=True)
            scatter(w, j).wait()
            gather(w + 3, j).start()
        return 0

    lax.fori_loop(0, (NW - 6) // 6, sextet, 0)
    for w in (NW - 3, NW - 2, NW - 1):
        gather(w, w % 6).wait()
        scatter(w, w % 6).start(add=True)
    for b in range(6):
        scatter(0, b).wait()

    plsc.subcore_barrier()

    pltpu.sync_copy(accum.at[pl.ds(r0, ROWS_PER_TEC)],
                    out_hbm.at[pl.ds(r0, ROWS_PER_TEC), pl.ds(ch0, CH)])


_CONV_SC_CACHE = []


def _conv_sc(z, src, dst):
    if not _CONV_SC_CACHE:
        _CONV_SC_CACHE.append(pl.kernel(
            _conv_sc_body,
            out_type=jax.ShapeDtypeStruct((N, C), jnp.float32),
            mesh=plsc.VectorSubcoreMesh(core_axis_name="c",
                                        subcore_axis_name="s"),
            scratch_types=[
                pltpu.VMEM_SHARED((ACC_ROWS, CH), jnp.float32),
                pltpu.VMEM((NW, K), jnp.int32),
                pltpu.VMEM((NW, K), jnp.int32),
                [pltpu.VMEM((K, CH), jnp.float32) for _ in range(6)],
                [pltpu.SemaphoreType.DMA for _ in range(6)],
                [pltpu.SemaphoreType.DMA for _ in range(6)],
            ],
            compiler_params=pltpu.CompilerParams(use_tc_tiling_on_sc=False),
        ))
    return _CONV_SC_CACHE[0](z, src, dst)


def kernel(x, edge_index, W1, gamma1, beta1, W2, gamma2, beta2):
    npad = E_PAD - E
    pad_i = jnp.arange(npad, dtype=jnp.int32)
    src = jnp.concatenate([edge_index[0], (pad_i * 131) % N]).reshape(
        NS, NW, K)
    dst = jnp.concatenate([edge_index[1], N + (pad_i % 8)]).reshape(NS, NW, K)

    w1s = jnp.stack([W1[:, :CH], W1[:, CH:]])
    w2s = jnp.stack([W2[:, :CH], W2[:, CH:]])

    z1 = _matmul_split(x, w1s)
    h = _conv_sc(z1, src, dst)
    stats1 = _bn_stats(h, gamma1, beta1)
    z2 = _affine_relu_matmul_split(h, stats1, w2s)
    h2 = _conv_sc(z2, src, dst)
    stats2 = _bn_stats(h2, gamma2, beta2)
    return _affine_residual_relu(h2, stats2, x)
